# baseline (speedup 1.0000x reference)
import functools

import jax
import jax.numpy as jnp
from jax import lax
from jax.experimental import pallas as pl
from jax.experimental.pallas import tpu as pltpu
from jax.experimental.pallas import tpu_sc as plsc

N, D, K, OUT = 160000, 256, 128, 64
NC, NS = 2, 16
NW = NC * NS
SPW = K // NW
CH = 128
NV = D // 16
DEC = 8
ND = N // DEC

_mesh = plsc.VectorSubcoreMesh(core_axis_name="c", subcore_axis_name="s",
                               num_cores=NC, num_subcores=NS)


@functools.partial(
    pl.kernel,
    out_type=[jax.ShapeDtypeStruct((NW, SPW, D), jnp.float32),
              jax.ShapeDtypeStruct((NW, 1, 4 * 16), jnp.float32)],
    mesh=_mesh,
    compiler_params=pltpu.CompilerParams(needs_layout_passes=False),
    scratch_types=[
        pltpu.VMEM((CH, D), jnp.float32),
        pltpu.VMEM((CH, D), jnp.float32),
        pltpu.VMEM((CH * 4,), jnp.float32),
        pltpu.VMEM((CH * 4,), jnp.float32),
        pltpu.VMEM((256,), jnp.int32),
        pltpu.VMEM((SPW + 1, 16), jnp.int32),
        pltpu.VMEM((SPW, D), jnp.float32),
        pltpu.VMEM((1, 4 * 16), jnp.float32),
        pltpu.SemaphoreType.DMA,
        pltpu.SemaphoreType.DMA,
        pltpu.SemaphoreType.DMA,
        pltpu.SemaphoreType.DMA,
    ],
)
def _sc_segment_sum(feat_hbm, ids_hbm, crd_hbm, jlo_hbm,
                    fsum_hbm, aux_hbm,
                    fbuf0, fbuf1, cbuf0, cbuf1, jlov, wbuf, ostage, austage,
                    sem0, sem1, csem0, csem1):
  c = lax.axis_index("c")
  sub = lax.axis_index("s")
  wid = c * NS + sub
  fbufs = (fbuf0, fbuf1)
  cbufs = (cbuf0, cbuf1)
  sems = (sem0, sem1)
  csems = (csem0, csem1)
  iota16 = lax.iota(jnp.int32, 16)

  pltpu.sync_copy(jlo_hbm, jlov)
  woffs = []
  for s in range(SPW + 1):
    k = SPW * wid + s
    jl = jlov[pl.ds(k, 16)][0]
    woff = pl.multiple_of(jnp.clip(DEC * jl - DEC, 0, N - 16), 8)
    woffs.append(woff)
    pltpu.async_copy(ids_hbm.at[pl.ds(woff, 16)], wbuf.at[s], sem0)
  for s in range(SPW + 1):
    pltpu.make_async_copy(ids_hbm.at[pl.ds(woffs[s], 16)],
                          wbuf.at[s], sem0).wait()
  bnd = []
  for s in range(SPW + 1):
    k = SPW * wid + s
    cnt = plsc.all_reduce_population_count(wbuf[s] < k)[0]
    bnd.append(woffs[s] + cnt)

  for s in range(SPW):
    rs = bnd[s]
    re = bnd[s + 1]
    a8 = (rs >> 3) << 3
    nch = lax.div(re - a8 + (CH - 1), CH)

    def chunk_start(g, slot, nch=nch, a8=a8):
      @pl.when(g < nch)
      def _():
        cs = pl.multiple_of(jnp.minimum(a8 + g * CH, N - CH), 8)
        pltpu.async_copy(feat_hbm.at[pl.ds(cs, CH)], fbufs[slot], sems[slot])
        pltpu.async_copy(crd_hbm.at[pl.ds(cs * 4, CH * 4)],
                         cbufs[slot], csems[slot])

    def chunk_wait(g, slot, nch=nch, a8=a8):
      @pl.when(g < nch)
      def _():
        cs = pl.multiple_of(jnp.minimum(a8 + g * CH, N - CH), 8)
        pltpu.make_async_copy(feat_hbm.at[pl.ds(cs, CH)],
                              fbufs[slot], sems[slot]).wait()
        pltpu.make_async_copy(crd_hbm.at[pl.ds(cs * 4, CH * 4)],
                              cbufs[slot], csems[slot]).wait()

    def chunk_rows(g, slot, carry, rs=rs, re=re, a8=a8):
      cs0 = a8 + g * CH
      cs = jnp.minimum(cs0, N - CH)
      lo = jnp.maximum(rs, cs0) - cs
      hi = jnp.minimum(re, cs0 + CH) - cs
      fb = fbufs[slot]
      cb = cbufs[slot]

      def row_body(r, accs):
        return tuple(accs[t] + fb[r, pl.ds(16 * t, 16)] for t in range(NV))

      accs = lax.fori_loop(lo, hi, row_body, carry[:NV])

      lo4 = lo * 4
      hi4 = hi * 4

      def aux_body(gi, aux):
        base = gi * 16
        gidx = base + iota16
        m = (gidx >= lo4) & (gidx < hi4)
        return aux + jnp.where(m, cb[pl.ds(base, 16)], 0.0)

      aux = lax.fori_loop(lo4 >> 4, (hi4 + 15) >> 4, aux_body, carry[NV])
      return accs + (aux,)

    def pair_body(j, carry):
      g0 = 2 * j
      chunk_start(g0 + 1, 1)
      chunk_wait(g0, 0)
      carry = chunk_rows(g0, 0, carry)
      chunk_start(g0 + 2, 0)
      chunk_wait(g0 + 1, 1)
      return chunk_rows(g0 + 1, 1, carry)

    chunk_start(0, 0)
    carry = lax.fori_loop(
        0, lax.div(nch + 1, 2), pair_body,
        tuple(jnp.zeros((16,), jnp.float32) for _ in range(NV + 1)))
    for t in range(NV):
      ostage[s, pl.ds(16 * t, 16)] = carry[t]
    austage[0, pl.ds(16 * s, 16)] = carry[NV]

  pltpu.sync_copy(ostage, fsum_hbm.at[wid])
  pltpu.sync_copy(austage, aux_hbm.at[wid])


def _tc_jlo_body(idsd_ref, jlo_ref):
  i = pl.program_id(0)

  @pl.when(i == 0)
  def _():
    jlo_ref[...] = jnp.zeros_like(jlo_ref)

  idsd = idsd_ref[0, 0]
  ltk = (idsd[:, None]
         < lax.broadcasted_iota(jnp.int32, (ND // 10, 256), 1)).astype(
             jnp.int32)
  jlo_ref[...] += jnp.sum(ltk, axis=0, keepdims=True)


def _tc_head_body(aux_ref, fs_ref, w1_ref, w2_ref, w3_ref, b3_ref,
                  emb_ref, cent_ref, out_ref):
  a = aux_ref[...]
  aux4 = a[:, 0:4] + a[:, 4:8] + a[:, 8:12] + a[:, 12:16]
  inv = 1.0 / jnp.maximum(aux4[:, 3:4], 1.0)
  emb = fs_ref[...] * inv
  emb_ref[...] = emb
  cent_ref[...] = aux4[:, 0:3] * inv
  h = jax.nn.relu(jnp.dot(emb, w1_ref[...],
                          preferred_element_type=jnp.float32))
  h = jax.nn.relu(jnp.dot(h, w2_ref[...],
                          preferred_element_type=jnp.float32))
  out_ref[...] = (jnp.dot(h, w3_ref[...],
                          preferred_element_type=jnp.float32) + b3_ref[...])


def kernel(features, coords, instance_ids, W1, W2, W3, b3):
  ids = instance_ids.astype(jnp.int32)
  idsd3 = ids[::DEC].reshape(10, 1, ND // 10)

  jlo2 = pl.pallas_call(
      _tc_jlo_body,
      grid=(10,),
      in_specs=[pl.BlockSpec((1, 1, ND // 10), lambda i: (i, 0, 0))],
      out_specs=pl.BlockSpec((1, 256), lambda i: (0, 0)),
      out_shape=jax.ShapeDtypeStruct((1, 256), jnp.int32),
  )(idsd3)

  crd4 = jnp.concatenate(
      [coords.astype(jnp.float32), jnp.ones((N, 1), jnp.float32)],
      axis=1).reshape(-1)
  fsum, aux = _sc_segment_sum(features, ids, crd4, jlo2[0])

  emb, cent, out = pl.pallas_call(
      _tc_head_body,
      out_shape=[jax.ShapeDtypeStruct((K, D), jnp.float32),
                 jax.ShapeDtypeStruct((K, 3), jnp.float32),
                 jax.ShapeDtypeStruct((K, OUT), jnp.float32)],
  )(aux.reshape(K, 16), fsum.reshape(K, D), W1, W2, W3, b3)
  return emb, cent, out


# speedup vs baseline: 1.1833x; 1.1833x over previous
import functools

import jax
import jax.numpy as jnp
from jax import lax
from jax.experimental import pallas as pl
from jax.experimental.pallas import tpu as pltpu
from jax.experimental.pallas import tpu_sc as plsc

N, D, K, OUT = 160000, 256, 128, 64
NC, NS = 2, 16
NW = NC * NS
SPW = K // NW
CH = 128
NV = D // 16
DEC = 8
ND = N // DEC

_mesh = plsc.VectorSubcoreMesh(core_axis_name="c", subcore_axis_name="s",
                               num_cores=NC, num_subcores=NS)


@functools.partial(
    pl.kernel,
    out_type=[jax.ShapeDtypeStruct((NW, SPW, D), jnp.float32),
              jax.ShapeDtypeStruct((NW, 1, 4 * 16), jnp.float32)],
    mesh=_mesh,
    compiler_params=pltpu.CompilerParams(needs_layout_passes=False),
    scratch_types=[
        pltpu.VMEM((CH, D), jnp.float32),
        pltpu.VMEM((CH, D), jnp.float32),
        pltpu.VMEM((CH * 3,), jnp.float32),
        pltpu.VMEM((CH * 3,), jnp.float32),
        pltpu.VMEM((256,), jnp.int32),
        pltpu.VMEM((SPW + 1, 16), jnp.int32),
        pltpu.VMEM((SPW, D), jnp.float32),
        pltpu.VMEM((1, 4 * 16), jnp.float32),
        pltpu.SemaphoreType.DMA,
        pltpu.SemaphoreType.DMA,
        pltpu.SemaphoreType.DMA,
        pltpu.SemaphoreType.DMA,
    ],
)
def _sc_segment_sum(feat_hbm, ids_hbm, crd_hbm, jlo_hbm,
                    fsum_hbm, aux_hbm,
                    fbuf0, fbuf1, cbuf0, cbuf1, jlov, wbuf, ostage, austage,
                    sem0, sem1, csem0, csem1):
  c = lax.axis_index("c")
  sub = lax.axis_index("s")
  wid = c * NS + sub
  fbufs = (fbuf0, fbuf1)
  cbufs = (cbuf0, cbuf1)
  sems = (sem0, sem1)
  csems = (csem0, csem1)
  iota16 = lax.iota(jnp.int32, 16)

  pltpu.sync_copy(jlo_hbm, jlov)
  woffs = []
  for s in range(SPW + 1):
    k = SPW * wid + s
    jl = jlov[pl.ds(k, 16)][0]
    woff = pl.multiple_of(jnp.clip(DEC * jl - DEC, 0, N - 16), 8)
    woffs.append(woff)
    pltpu.async_copy(ids_hbm.at[pl.ds(woff, 16)], wbuf.at[s], sem0)
  for s in range(SPW + 1):
    pltpu.make_async_copy(ids_hbm.at[pl.ds(woffs[s], 16)],
                          wbuf.at[s], sem0).wait()
  bnd = []
  for s in range(SPW + 1):
    k = SPW * wid + s
    cnt = plsc.all_reduce_population_count(wbuf[s] < k)[0]
    bnd.append(woffs[s] + cnt)

  for s in range(SPW):
    rs = bnd[s]
    re = bnd[s + 1]
    a8 = (rs >> 3) << 3
    nch = lax.div(re - a8 + (CH - 1), CH)

    def chunk_start(g, slot, nch=nch, a8=a8):
      @pl.when(g < nch)
      def _():
        cs = pl.multiple_of(jnp.minimum(a8 + g * CH, N - CH), 8)
        pltpu.async_copy(feat_hbm.at[pl.ds(cs, CH)], fbufs[slot], sems[slot])
        pltpu.async_copy(crd_hbm.at[pl.ds(cs * 3, CH * 3)],
                         cbufs[slot], csems[slot])

    def chunk_wait(g, slot, nch=nch, a8=a8):
      @pl.when(g < nch)
      def _():
        cs = pl.multiple_of(jnp.minimum(a8 + g * CH, N - CH), 8)
        pltpu.make_async_copy(feat_hbm.at[pl.ds(cs, CH)],
                              fbufs[slot], sems[slot]).wait()
        pltpu.make_async_copy(crd_hbm.at[pl.ds(cs * 3, CH * 3)],
                              cbufs[slot], csems[slot]).wait()

    def chunk_rows(g, slot, carry, rs=rs, re=re, a8=a8):
      cs0 = a8 + g * CH
      cs = jnp.minimum(cs0, N - CH)
      lo = jnp.maximum(rs, cs0) - cs
      hi = jnp.minimum(re, cs0 + CH) - cs
      fb = fbufs[slot]
      cb = cbufs[slot]

      def row_body(r, accs):
        return tuple(accs[t] + fb[r, pl.ds(16 * t, 16)] for t in range(NV))

      accs = lax.fori_loop(lo, hi, row_body, carry[:NV])

      lo3 = lo * 3
      hi3 = hi * 3

      def aux_body(gi, aux):
        base = gi * 16
        gidx = base + iota16
        inr = (gidx >= lo3) & (gidx < hi3)
        gm = lax.rem(gidx, 3)
        v = cb[pl.ds(base, 16)]
        zero = jnp.zeros((16,), jnp.float32)
        return tuple(aux[cc] + jnp.where(inr & (gm == cc), v, zero)
                     for cc in range(3))

      aux = lax.fori_loop(lo3 >> 4, (hi3 + 15) >> 4, aux_body,
                          (carry[NV], carry[NV + 1], carry[NV + 2]))
      return accs + aux

    def pair_body(j, carry):
      g0 = 2 * j
      chunk_start(g0 + 1, 1)
      chunk_wait(g0, 0)
      carry = chunk_rows(g0, 0, carry)
      chunk_start(g0 + 2, 0)
      chunk_wait(g0 + 1, 1)
      return chunk_rows(g0 + 1, 1, carry)

    chunk_start(0, 0)
    carry = lax.fori_loop(
        0, lax.div(nch + 1, 2), pair_body,
        tuple(jnp.zeros((16,), jnp.float32) for _ in range(NV + 3)))
    for t in range(NV):
      ostage[s, pl.ds(16 * t, 16)] = carry[t]
    av = jnp.where(iota16 == 3, (re - rs).astype(jnp.float32), 0.0)
    for cc in range(3):
      av = av + jnp.where(iota16 == cc, jnp.sum(carry[NV + cc]), 0.0)
    austage[0, pl.ds(16 * s, 16)] = av

  pltpu.sync_copy(ostage, fsum_hbm.at[wid])
  pltpu.sync_copy(austage, aux_hbm.at[wid])


def _tc_jlo_body(idsd_ref, jlo_ref):
  i = pl.program_id(0)

  @pl.when(i == 0)
  def _():
    jlo_ref[...] = jnp.zeros_like(jlo_ref)

  idsd = idsd_ref[0, 0]
  ltk = (idsd[:, None]
         < lax.broadcasted_iota(jnp.int32, (ND // 10, 256), 1)).astype(
             jnp.int32)
  jlo_ref[...] += jnp.sum(ltk, axis=0, keepdims=True)


def _tc_head_body(aux_ref, fs_ref, w1_ref, w2_ref, w3_ref, b3_ref,
                  emb_ref, cent_ref, out_ref):
  a = aux_ref[...]
  aux4 = a[:, 0:4]
  inv = 1.0 / jnp.maximum(aux4[:, 3:4], 1.0)
  emb = fs_ref[...] * inv
  emb_ref[...] = emb
  cent_ref[...] = aux4[:, 0:3] * inv
  h = jax.nn.relu(jnp.dot(emb, w1_ref[...],
                          preferred_element_type=jnp.float32))
  h = jax.nn.relu(jnp.dot(h, w2_ref[...],
                          preferred_element_type=jnp.float32))
  out_ref[...] = (jnp.dot(h, w3_ref[...],
                          preferred_element_type=jnp.float32) + b3_ref[...])


def kernel(features, coords, instance_ids, W1, W2, W3, b3):
  ids = instance_ids.astype(jnp.int32)
  idsd3 = ids[::DEC].reshape(10, 1, ND // 10)

  jlo2 = pl.pallas_call(
      _tc_jlo_body,
      grid=(10,),
      in_specs=[pl.BlockSpec((1, 1, ND // 10), lambda i: (i, 0, 0))],
      out_specs=pl.BlockSpec((1, 256), lambda i: (0, 0)),
      out_shape=jax.ShapeDtypeStruct((1, 256), jnp.int32),
  )(idsd3)

  cflat = coords.astype(jnp.float32).reshape(-1)
  fsum, aux = _sc_segment_sum(features, ids, cflat, jlo2[0])

  emb, cent, out = pl.pallas_call(
      _tc_head_body,
      out_shape=[jax.ShapeDtypeStruct((K, D), jnp.float32),
                 jax.ShapeDtypeStruct((K, 3), jnp.float32),
                 jax.ShapeDtypeStruct((K, OUT), jnp.float32)],
  )(aux.reshape(K, 16), fsum.reshape(K, D), W1, W2, W3, b3)
  return emb, cent, out


# speedup vs baseline: 1.8816x; 1.5901x over previous
import functools

import jax
import jax.numpy as jnp
from jax import lax
from jax.experimental import pallas as pl
from jax.experimental.pallas import tpu as pltpu
from jax.experimental.pallas import tpu_sc as plsc

N, D, K, OUT = 160000, 256, 128, 64
NC, NS = 2, 16
NW = NC * NS
SPW = K // NW
CH = 128
NV = D // 16
DEC = 8
ND = N // DEC

_mesh = plsc.VectorSubcoreMesh(core_axis_name="c", subcore_axis_name="s",
                               num_cores=NC, num_subcores=NS)


@functools.partial(
    pl.kernel,
    out_type=jax.ShapeDtypeStruct((NW, SPW, D), jnp.float32),
    mesh=_mesh,
    compiler_params=pltpu.CompilerParams(needs_layout_passes=False),
    scratch_types=[
        pltpu.VMEM((CH, D), jnp.float32),
        pltpu.VMEM((CH, D), jnp.float32),
        pltpu.VMEM((256,), jnp.int32),
        pltpu.VMEM((SPW + 1, 16), jnp.int32),
        pltpu.VMEM((SPW, D), jnp.float32),
        pltpu.SemaphoreType.DMA,
        pltpu.SemaphoreType.DMA,
    ],
)
def _sc_segment_sum(feat_hbm, ids_hbm, jlo_hbm, fsum_hbm,
                    fbuf0, fbuf1, jlov, wbuf, ostage, sem0, sem1):
  c = lax.axis_index("c")
  sub = lax.axis_index("s")
  wid = c * NS + sub
  fbufs = (fbuf0, fbuf1)
  sems = (sem0, sem1)

  pltpu.sync_copy(jlo_hbm, jlov)
  woffs = []
  for s in range(SPW + 1):
    k = SPW * wid + s
    jl = jlov[pl.ds(k, 16)][0]
    woff = pl.multiple_of(jnp.clip(DEC * jl - DEC, 0, N - 16), 8)
    woffs.append(woff)
    pltpu.async_copy(ids_hbm.at[pl.ds(woff, 16)], wbuf.at[s], sem0)
  for s in range(SPW + 1):
    pltpu.make_async_copy(ids_hbm.at[pl.ds(woffs[s], 16)],
                          wbuf.at[s], sem0).wait()
  bnd = []
  for s in range(SPW + 1):
    k = SPW * wid + s
    cnt = plsc.all_reduce_population_count(wbuf[s] < k)[0]
    bnd.append(woffs[s] + cnt)

  for s in range(SPW):
    rs = bnd[s]
    re = bnd[s + 1]
    a8 = (rs >> 3) << 3
    nch = lax.div(re - a8 + (CH - 1), CH)

    def chunk_start(g, slot, nch=nch, a8=a8):
      @pl.when(g < nch)
      def _():
        cs = pl.multiple_of(jnp.minimum(a8 + g * CH, N - CH), 8)
        pltpu.async_copy(feat_hbm.at[pl.ds(cs, CH)], fbufs[slot], sems[slot])

    def chunk_wait(g, slot, nch=nch, a8=a8):
      @pl.when(g < nch)
      def _():
        cs = pl.multiple_of(jnp.minimum(a8 + g * CH, N - CH), 8)
        pltpu.make_async_copy(feat_hbm.at[pl.ds(cs, CH)],
                              fbufs[slot], sems[slot]).wait()

    def chunk_rows(g, slot, carry, rs=rs, re=re, a8=a8):
      cs0 = a8 + g * CH
      cs = jnp.minimum(cs0, N - CH)
      lo = jnp.maximum(rs, cs0) - cs
      hi = jnp.minimum(re, cs0 + CH) - cs
      fb = fbufs[slot]

      def row_body(r, accs):
        return tuple(accs[t] + fb[r, pl.ds(16 * t, 16)] for t in range(NV))

      return lax.fori_loop(lo, hi, row_body, carry)

    def pair_body(j, carry):
      g0 = 2 * j
      chunk_start(g0 + 1, 1)
      chunk_wait(g0, 0)
      carry = chunk_rows(g0, 0, carry)
      chunk_start(g0 + 2, 0)
      chunk_wait(g0 + 1, 1)
      return chunk_rows(g0 + 1, 1, carry)

    chunk_start(0, 0)
    carry = lax.fori_loop(
        0, lax.div(nch + 1, 2), pair_body,
        tuple(jnp.zeros((16,), jnp.float32) for _ in range(NV)))
    for t in range(NV):
      ostage[s, pl.ds(16 * t, 16)] = carry[t]

  pltpu.sync_copy(ostage, fsum_hbm.at[wid])


def _tc_jlo_body(idsd_ref, jlo_ref):
  i = pl.program_id(0)

  @pl.when(i == 0)
  def _():
    jlo_ref[...] = jnp.zeros_like(jlo_ref)

  idsd = idsd_ref[0, 0]
  ltk = (idsd[:, None]
         < lax.broadcasted_iota(jnp.int32, (ND // 10, 256), 1)).astype(
             jnp.int32)
  jlo_ref[...] += jnp.sum(ltk, axis=0, keepdims=True)


BN2 = 4000
NB2 = N // BN2


def _tc_aux_body(ids_ref, c3_ref, aux_ref):
  i = pl.program_id(0)

  @pl.when(i == 0)
  def _():
    aux_ref[...] = jnp.zeros_like(aux_ref)

  ids = ids_ref[0, 0]
  oh = (lax.broadcasted_iota(jnp.int32, (K, BN2), 0)
        == ids[None, :]).astype(jnp.float32)
  csum = lax.dot_general(oh, c3_ref[...], (((1,), (0,)), ((), ())),
                         preferred_element_type=jnp.float32)
  cnt = jnp.sum(oh, axis=1, keepdims=True)
  aux_ref[...] += jnp.concatenate([csum, cnt], axis=1)


def _tc_head_body(aux_ref, fs_ref, w1_ref, w2_ref, w3_ref, b3_ref,
                  emb_ref, cent_ref, out_ref):
  aux4 = aux_ref[...]
  inv = 1.0 / jnp.maximum(aux4[:, 3:4], 1.0)
  emb = fs_ref[...] * inv
  emb_ref[...] = emb
  cent_ref[...] = aux4[:, 0:3] * inv
  h = jax.nn.relu(jnp.dot(emb, w1_ref[...],
                          preferred_element_type=jnp.float32))
  h = jax.nn.relu(jnp.dot(h, w2_ref[...],
                          preferred_element_type=jnp.float32))
  out_ref[...] = (jnp.dot(h, w3_ref[...],
                          preferred_element_type=jnp.float32) + b3_ref[...])


def kernel(features, coords, instance_ids, W1, W2, W3, b3):
  ids = instance_ids.astype(jnp.int32)
  idsd3 = ids[::DEC].reshape(10, 1, ND // 10)

  jlo2 = pl.pallas_call(
      _tc_jlo_body,
      grid=(10,),
      in_specs=[pl.BlockSpec((1, 1, ND // 10), lambda i: (i, 0, 0))],
      out_specs=pl.BlockSpec((1, 256), lambda i: (0, 0)),
      out_shape=jax.ShapeDtypeStruct((1, 256), jnp.int32),
  )(idsd3)

  fsum = _sc_segment_sum(features, ids, jlo2[0])

  aux4 = pl.pallas_call(
      _tc_aux_body,
      grid=(NB2,),
      in_specs=[
          pl.BlockSpec((1, 1, BN2), lambda i: (i, 0, 0)),
          pl.BlockSpec((BN2, 3), lambda i: (i, 0)),
      ],
      out_specs=pl.BlockSpec((K, 4), lambda i: (0, 0)),
      out_shape=jax.ShapeDtypeStruct((K, 4), jnp.float32),
  )(ids.reshape(NB2, 1, BN2), coords.astype(jnp.float32))

  emb, cent, out = pl.pallas_call(
      _tc_head_body,
      out_shape=[jax.ShapeDtypeStruct((K, D), jnp.float32),
                 jax.ShapeDtypeStruct((K, 3), jnp.float32),
                 jax.ShapeDtypeStruct((K, OUT), jnp.float32)],
  )(aux4, fsum.reshape(K, D), W1, W2, W3, b3)
  return emb, cent, out


# speedup vs baseline: 1.8904x; 1.0047x over previous
import functools

import jax
import jax.numpy as jnp
from jax import lax
from jax.experimental import pallas as pl
from jax.experimental.pallas import tpu as pltpu
from jax.experimental.pallas import tpu_sc as plsc

N, D, K, OUT = 160000, 256, 128, 64
NC, NS = 2, 16
NW = NC * NS
SPW = K // NW
CH = 128
NV = D // 16
DEC = 8
ND = N // DEC

_mesh = plsc.VectorSubcoreMesh(core_axis_name="c", subcore_axis_name="s",
                               num_cores=NC, num_subcores=NS)


@functools.partial(
    pl.kernel,
    out_type=jax.ShapeDtypeStruct((NW, SPW, D), jnp.float32),
    mesh=_mesh,
    compiler_params=pltpu.CompilerParams(needs_layout_passes=False),
    scratch_types=[
        pltpu.VMEM((CH, D), jnp.float32),
        pltpu.VMEM((CH, D), jnp.float32),
        pltpu.VMEM((256,), jnp.int32),
        pltpu.VMEM((SPW + 1, 16), jnp.int32),
        pltpu.VMEM((SPW, D), jnp.float32),
        pltpu.SemaphoreType.DMA,
        pltpu.SemaphoreType.DMA,
    ],
)
def _sc_segment_sum(feat_hbm, ids_hbm, jlo_hbm, fsum_hbm,
                    fbuf0, fbuf1, jlov, wbuf, ostage, sem0, sem1):
  c = lax.axis_index("c")
  sub = lax.axis_index("s")
  wid = c * NS + sub
  fbufs = (fbuf0, fbuf1)
  sems = (sem0, sem1)

  pltpu.sync_copy(jlo_hbm, jlov)
  woffs = []
  for s in range(SPW + 1):
    k = SPW * wid + s
    jl = jlov[pl.ds(k, 16)][0]
    woff = pl.multiple_of(jnp.clip(DEC * jl - DEC, 0, N - 16), 8)
    woffs.append(woff)
    pltpu.async_copy(ids_hbm.at[pl.ds(woff, 16)], wbuf.at[s], sem0)
  for s in range(SPW + 1):
    pltpu.make_async_copy(ids_hbm.at[pl.ds(woffs[s], 16)],
                          wbuf.at[s], sem0).wait()
  bnd = []
  for s in range(SPW + 1):
    k = SPW * wid + s
    cnt = plsc.all_reduce_population_count(wbuf[s] < k)[0]
    bnd.append(woffs[s] + cnt)

  for s in range(SPW):
    for t in range(NV):
      ostage[s, pl.ds(16 * t, 16)] = jnp.zeros((16,), jnp.float32)

  a8 = (bnd[0] >> 3) << 3
  nch = lax.div(bnd[SPW] - a8 + (CH - 1), CH)

  def chunk_start(g, slot):
    @pl.when(g < nch)
    def _():
      cs = pl.multiple_of(jnp.minimum(a8 + g * CH, N - CH), 8)
      pltpu.async_copy(feat_hbm.at[pl.ds(cs, CH)], fbufs[slot], sems[slot])

  def chunk_wait(g, slot):
    @pl.when(g < nch)
    def _():
      cs = pl.multiple_of(jnp.minimum(a8 + g * CH, N - CH), 8)
      pltpu.make_async_copy(feat_hbm.at[pl.ds(cs, CH)],
                            fbufs[slot], sems[slot]).wait()

  def chunk_rows(g, slot):
    cs0 = a8 + g * CH
    cs = jnp.minimum(cs0, N - CH)
    fb = fbufs[slot]
    for s in range(SPW):
      lo = jnp.maximum(bnd[s], cs0) - cs
      hi = jnp.minimum(bnd[s + 1], cs0 + CH) - cs

      @pl.when(lo < hi)
      def _(s=s, lo=lo, hi=hi):
        def row_body(r, accs):
          return tuple(accs[t] + fb[r, pl.ds(16 * t, 16)] for t in range(NV))

        carry = lax.fori_loop(
            lo, hi, row_body,
            tuple(jnp.zeros((16,), jnp.float32) for _ in range(NV)))
        for t in range(NV):
          ostage[s, pl.ds(16 * t, 16)] += carry[t]

  def pair_body(j, _):
    g0 = 2 * j
    chunk_start(g0 + 1, 1)
    chunk_wait(g0, 0)
    chunk_rows(g0, 0)
    chunk_start(g0 + 2, 0)
    chunk_wait(g0 + 1, 1)
    chunk_rows(g0 + 1, 1)
    return 0

  chunk_start(0, 0)
  lax.fori_loop(0, lax.div(nch + 1, 2), pair_body, 0)

  pltpu.sync_copy(ostage, fsum_hbm.at[wid])


def _tc_jlo_body(idsd_ref, jlo_ref):
  i = pl.program_id(0)

  @pl.when(i == 0)
  def _():
    jlo_ref[...] = jnp.zeros_like(jlo_ref)

  idsd = idsd_ref[0, 0]
  ltk = (idsd[:, None]
         < lax.broadcasted_iota(jnp.int32, (ND // 10, 256), 1)).astype(
             jnp.int32)
  jlo_ref[...] += jnp.sum(ltk, axis=0, keepdims=True)


BN2 = 4000
NB2 = N // BN2


def _tc_aux_body(ids_ref, c3_ref, aux_ref):
  i = pl.program_id(0)

  @pl.when(i == 0)
  def _():
    aux_ref[...] = jnp.zeros_like(aux_ref)

  ids = ids_ref[0, 0]
  oh = (lax.broadcasted_iota(jnp.int32, (K, BN2), 0)
        == ids[None, :]).astype(jnp.float32)
  csum = lax.dot_general(oh, c3_ref[...], (((1,), (0,)), ((), ())),
                         preferred_element_type=jnp.float32)
  cnt = jnp.sum(oh, axis=1, keepdims=True)
  aux_ref[...] += jnp.concatenate([csum, cnt], axis=1)


def _tc_head_body(aux_ref, fs_ref, w1_ref, w2_ref, w3_ref, b3_ref,
                  emb_ref, cent_ref, out_ref):
  aux4 = aux_ref[...]
  inv = 1.0 / jnp.maximum(aux4[:, 3:4], 1.0)
  emb = fs_ref[...] * inv
  emb_ref[...] = emb
  cent_ref[...] = aux4[:, 0:3] * inv
  h = jax.nn.relu(jnp.dot(emb, w1_ref[...],
                          preferred_element_type=jnp.float32))
  h = jax.nn.relu(jnp.dot(h, w2_ref[...],
                          preferred_element_type=jnp.float32))
  out_ref[...] = (jnp.dot(h, w3_ref[...],
                          preferred_element_type=jnp.float32) + b3_ref[...])


def kernel(features, coords, instance_ids, W1, W2, W3, b3):
  ids = instance_ids.astype(jnp.int32)
  idsd3 = ids[::DEC].reshape(10, 1, ND // 10)

  jlo2 = pl.pallas_call(
      _tc_jlo_body,
      grid=(10,),
      in_specs=[pl.BlockSpec((1, 1, ND // 10), lambda i: (i, 0, 0))],
      out_specs=pl.BlockSpec((1, 256), lambda i: (0, 0)),
      out_shape=jax.ShapeDtypeStruct((1, 256), jnp.int32),
  )(idsd3)

  fsum = _sc_segment_sum(features, ids, jlo2[0])

  aux4 = pl.pallas_call(
      _tc_aux_body,
      grid=(NB2,),
      in_specs=[
          pl.BlockSpec((1, 1, BN2), lambda i: (i, 0, 0)),
          pl.BlockSpec((BN2, 3), lambda i: (i, 0)),
      ],
      out_specs=pl.BlockSpec((K, 4), lambda i: (0, 0)),
      out_shape=jax.ShapeDtypeStruct((K, 4), jnp.float32),
  )(ids.reshape(NB2, 1, BN2), coords.astype(jnp.float32))

  emb, cent, out = pl.pallas_call(
      _tc_head_body,
      out_shape=[jax.ShapeDtypeStruct((K, D), jnp.float32),
                 jax.ShapeDtypeStruct((K, 3), jnp.float32),
                 jax.ShapeDtypeStruct((K, OUT), jnp.float32)],
  )(aux4, fsum.reshape(K, D), W1, W2, W3, b3)
  return emb, cent, out


# speedup vs baseline: 1.8905x; 1.0001x over previous
import functools

import jax
import jax.numpy as jnp
from jax import lax
from jax.experimental import pallas as pl
from jax.experimental.pallas import tpu as pltpu
from jax.experimental.pallas import tpu_sc as plsc

N, D, K, OUT = 160000, 256, 128, 64
NC, NS = 2, 16
NW = NC * NS
SPW = K // NW
CH = 128
NV = D // 16
DEC = 8
ND = N // DEC

_mesh = plsc.VectorSubcoreMesh(core_axis_name="c", subcore_axis_name="s",
                               num_cores=NC, num_subcores=NS)


@functools.partial(
    pl.kernel,
    out_type=jax.ShapeDtypeStruct((NW, SPW, D), jnp.float32),
    mesh=_mesh,
    compiler_params=pltpu.CompilerParams(needs_layout_passes=False),
    scratch_types=[
        pltpu.VMEM((CH, D), jnp.float32),
        pltpu.VMEM((CH, D), jnp.float32),
        pltpu.VMEM((256,), jnp.int32),
        pltpu.VMEM((SPW + 1, 16), jnp.int32),
        pltpu.VMEM((SPW, D), jnp.float32),
        pltpu.SemaphoreType.DMA,
        pltpu.SemaphoreType.DMA,
    ],
)
def _sc_segment_sum(feat_hbm, ids_hbm, jlo_hbm, fsum_hbm,
                    fbuf0, fbuf1, jlov, wbuf, ostage, sem0, sem1):
  c = lax.axis_index("c")
  sub = lax.axis_index("s")
  wid = c * NS + sub
  fbufs = (fbuf0, fbuf1)
  sems = (sem0, sem1)

  pltpu.sync_copy(jlo_hbm, jlov)
  woffs = []
  for s in range(SPW + 1):
    k = SPW * wid + s
    jl = jlov[pl.ds(k, 16)][0]
    woff = pl.multiple_of(jnp.clip(DEC * jl - DEC, 0, N - 16), 8)
    woffs.append(woff)
    pltpu.async_copy(ids_hbm.at[pl.ds(woff, 16)], wbuf.at[s], sem0)
  for s in range(SPW + 1):
    pltpu.make_async_copy(ids_hbm.at[pl.ds(woffs[s], 16)],
                          wbuf.at[s], sem0).wait()
  bnd = []
  for s in range(SPW + 1):
    k = SPW * wid + s
    cnt = plsc.all_reduce_population_count(wbuf[s] < k)[0]
    bnd.append(woffs[s] + cnt)

  for s in range(SPW):
    for t in range(NV):
      ostage[s, pl.ds(16 * t, 16)] = jnp.zeros((16,), jnp.float32)

  a8 = (bnd[0] >> 3) << 3
  nch = lax.div(bnd[SPW] - a8 + (CH - 1), CH)

  def chunk_start(g, slot):
    @pl.when(g < nch)
    def _():
      cs = pl.multiple_of(jnp.minimum(a8 + g * CH, N - CH), 8)
      pltpu.async_copy(feat_hbm.at[pl.ds(cs, CH)], fbufs[slot], sems[slot])

  def chunk_wait(g, slot):
    @pl.when(g < nch)
    def _():
      cs = pl.multiple_of(jnp.minimum(a8 + g * CH, N - CH), 8)
      pltpu.make_async_copy(feat_hbm.at[pl.ds(cs, CH)],
                            fbufs[slot], sems[slot]).wait()

  def chunk_rows(g, slot):
    cs0 = a8 + g * CH
    cs = jnp.minimum(cs0, N - CH)
    fb = fbufs[slot]
    for s in range(SPW):
      lo = jnp.maximum(bnd[s], cs0) - cs
      hi = jnp.minimum(bnd[s + 1], cs0 + CH) - cs

      @pl.when(lo < hi)
      def _(s=s, lo=lo, hi=hi):
        n4 = lo + (((hi - lo) >> 2) << 2)

        def quad_body(r0, accs):
          r = lo + 4 * r0
          return tuple(
              ((accs[t] + fb[r, pl.ds(16 * t, 16)]
                + fb[r + 1, pl.ds(16 * t, 16)])
               + (fb[r + 2, pl.ds(16 * t, 16)]
                  + fb[r + 3, pl.ds(16 * t, 16)]))
              for t in range(NV))

        def row_body(r, accs):
          return tuple(accs[t] + fb[r, pl.ds(16 * t, 16)] for t in range(NV))

        carry = lax.fori_loop(
            0, (hi - lo) >> 2, quad_body,
            tuple(jnp.zeros((16,), jnp.float32) for _ in range(NV)))
        carry = lax.fori_loop(n4, hi, row_body, carry)
        for t in range(NV):
          ostage[s, pl.ds(16 * t, 16)] += carry[t]

  def pair_body(j, _):
    g0 = 2 * j
    chunk_start(g0 + 1, 1)
    chunk_wait(g0, 0)
    chunk_rows(g0, 0)
    chunk_start(g0 + 2, 0)
    chunk_wait(g0 + 1, 1)
    chunk_rows(g0 + 1, 1)
    return 0

  chunk_start(0, 0)
  lax.fori_loop(0, lax.div(nch + 1, 2), pair_body, 0)

  pltpu.sync_copy(ostage, fsum_hbm.at[wid])


def _tc_jlo_body(idsd_ref, jlo_ref):
  i = pl.program_id(0)

  @pl.when(i == 0)
  def _():
    jlo_ref[...] = jnp.zeros_like(jlo_ref)

  idsd = idsd_ref[0, 0]
  ltk = (idsd[:, None]
         < lax.broadcasted_iota(jnp.int32, (ND // 10, 256), 1)).astype(
             jnp.int32)
  jlo_ref[...] += jnp.sum(ltk, axis=0, keepdims=True)


BN2 = 4000
NB2 = N // BN2


def _tc_aux_body(ids_ref, c3_ref, aux_ref):
  i = pl.program_id(0)

  @pl.when(i == 0)
  def _():
    aux_ref[...] = jnp.zeros_like(aux_ref)

  ids = ids_ref[0, 0]
  oh = (lax.broadcasted_iota(jnp.int32, (K, BN2), 0)
        == ids[None, :]).astype(jnp.float32)
  csum = lax.dot_general(oh, c3_ref[...], (((1,), (0,)), ((), ())),
                         preferred_element_type=jnp.float32)
  cnt = jnp.sum(oh, axis=1, keepdims=True)
  aux_ref[...] += jnp.concatenate([csum, cnt], axis=1)


def _tc_head_body(aux_ref, fs_ref, w1_ref, w2_ref, w3_ref, b3_ref,
                  emb_ref, cent_ref, out_ref):
  aux4 = aux_ref[...]
  inv = 1.0 / jnp.maximum(aux4[:, 3:4], 1.0)
  emb = fs_ref[...] * inv
  emb_ref[...] = emb
  cent_ref[...] = aux4[:, 0:3] * inv
  h = jax.nn.relu(jnp.dot(emb, w1_ref[...],
                          preferred_element_type=jnp.float32))
  h = jax.nn.relu(jnp.dot(h, w2_ref[...],
                          preferred_element_type=jnp.float32))
  out_ref[...] = (jnp.dot(h, w3_ref[...],
                          preferred_element_type=jnp.float32) + b3_ref[...])


def kernel(features, coords, instance_ids, W1, W2, W3, b3):
  ids = instance_ids.astype(jnp.int32)
  idsd3 = ids[::DEC].reshape(10, 1, ND // 10)

  jlo2 = pl.pallas_call(
      _tc_jlo_body,
      grid=(10,),
      in_specs=[pl.BlockSpec((1, 1, ND // 10), lambda i: (i, 0, 0))],
      out_specs=pl.BlockSpec((1, 256), lambda i: (0, 0)),
      out_shape=jax.ShapeDtypeStruct((1, 256), jnp.int32),
  )(idsd3)

  fsum = _sc_segment_sum(features, ids, jlo2[0])

  aux4 = pl.pallas_call(
      _tc_aux_body,
      grid=(NB2,),
      in_specs=[
          pl.BlockSpec((1, 1, BN2), lambda i: (i, 0, 0)),
          pl.BlockSpec((BN2, 3), lambda i: (i, 0)),
      ],
      out_specs=pl.BlockSpec((K, 4), lambda i: (0, 0)),
      out_shape=jax.ShapeDtypeStruct((K, 4), jnp.float32),
  )(ids.reshape(NB2, 1, BN2), coords.astype(jnp.float32))

  emb, cent, out = pl.pallas_call(
      _tc_head_body,
      out_shape=[jax.ShapeDtypeStruct((K, D), jnp.float32),
                 jax.ShapeDtypeStruct((K, 3), jnp.float32),
                 jax.ShapeDtypeStruct((K, OUT), jnp.float32)],
  )(aux4, fsum.reshape(K, D), W1, W2, W3, b3)
  return emb, cent, out


# speedup vs baseline: 1.9820x; 1.0484x over previous
import functools

import jax
import jax.numpy as jnp
from jax import lax
from jax.experimental import pallas as pl
from jax.experimental.pallas import tpu as pltpu
from jax.experimental.pallas import tpu_sc as plsc

N, D, K, OUT = 160000, 256, 128, 64
NC, NS = 2, 16
NW = NC * NS
SPW = K // NW
CH = 128
NV = D // 16
DEC = 8
ND = N // DEC

_mesh = plsc.VectorSubcoreMesh(core_axis_name="c", subcore_axis_name="s",
                               num_cores=NC, num_subcores=NS)


@functools.partial(
    pl.kernel,
    out_type=jax.ShapeDtypeStruct((NW, SPW, D), jnp.float32),
    mesh=_mesh,
    compiler_params=pltpu.CompilerParams(needs_layout_passes=False),
    scratch_types=[
        pltpu.VMEM((CH, D), jnp.float32),
        pltpu.VMEM((CH, D), jnp.float32),
        pltpu.VMEM((ND + 16,), jnp.int32),
        pltpu.VMEM((SPW + 1, 16), jnp.int32),
        pltpu.VMEM((SPW, D), jnp.float32),
        pltpu.SemaphoreType.DMA,
        pltpu.SemaphoreType.DMA,
    ],
)
def _sc_segment_sum(feat_hbm, ids_hbm, idsd_hbm, fsum_hbm,
                    fbuf0, fbuf1, idsv, wbuf, ostage, sem0, sem1):
  c = lax.axis_index("c")
  sub = lax.axis_index("s")
  wid = c * NS + sub
  fbufs = (fbuf0, fbuf1)
  sems = (sem0, sem1)

  pltpu.sync_copy(idsd_hbm, idsv.at[pl.ds(0, ND)])
  woffs = []
  for s in range(SPW + 1):
    k = SPW * wid + s

    def bs_body(_, lohi, k=k):
      lo, hi = lohi
      mid = (lo + hi) >> 1
      v = idsv[pl.ds(mid, 16)][0]
      return jnp.where(v < k, mid + 1, lo), jnp.where(v < k, hi, mid)

    jl, _ = lax.fori_loop(0, 15, bs_body, (jnp.int32(0), jnp.int32(ND)))
    woff = pl.multiple_of(jnp.clip(DEC * jl - DEC, 0, N - 16), 8)
    woffs.append(woff)
    pltpu.async_copy(ids_hbm.at[pl.ds(woff, 16)], wbuf.at[s], sem0)
  for s in range(SPW + 1):
    pltpu.make_async_copy(ids_hbm.at[pl.ds(woffs[s], 16)],
                          wbuf.at[s], sem0).wait()
  bnd = []
  for s in range(SPW + 1):
    k = SPW * wid + s
    cnt = plsc.all_reduce_population_count(wbuf[s] < k)[0]
    bnd.append(woffs[s] + cnt)

  for s in range(SPW):
    for t in range(NV):
      ostage[s, pl.ds(16 * t, 16)] = jnp.zeros((16,), jnp.float32)

  a8 = (bnd[0] >> 3) << 3
  nch = lax.div(bnd[SPW] - a8 + (CH - 1), CH)

  def chunk_start(g, slot):
    @pl.when(g < nch)
    def _():
      cs = pl.multiple_of(jnp.minimum(a8 + g * CH, N - CH), 8)
      pltpu.async_copy(feat_hbm.at[pl.ds(cs, CH)], fbufs[slot], sems[slot])

  def chunk_wait(g, slot):
    @pl.when(g < nch)
    def _():
      cs = pl.multiple_of(jnp.minimum(a8 + g * CH, N - CH), 8)
      pltpu.make_async_copy(feat_hbm.at[pl.ds(cs, CH)],
                            fbufs[slot], sems[slot]).wait()

  def chunk_rows(g, slot):
    cs0 = a8 + g * CH
    cs = jnp.minimum(cs0, N - CH)
    fb = fbufs[slot]
    for s in range(SPW):
      lo = jnp.maximum(bnd[s], cs0) - cs
      hi = jnp.minimum(bnd[s + 1], cs0 + CH) - cs

      @pl.when(lo < hi)
      def _(s=s, lo=lo, hi=hi):
        n4 = lo + (((hi - lo) >> 2) << 2)

        def quad_body(r0, accs):
          r = lo + 4 * r0
          return tuple(
              ((accs[t] + fb[r, pl.ds(16 * t, 16)]
                + fb[r + 1, pl.ds(16 * t, 16)])
               + (fb[r + 2, pl.ds(16 * t, 16)]
                  + fb[r + 3, pl.ds(16 * t, 16)]))
              for t in range(NV))

        def row_body(r, accs):
          return tuple(accs[t] + fb[r, pl.ds(16 * t, 16)] for t in range(NV))

        carry = lax.fori_loop(
            0, (hi - lo) >> 2, quad_body,
            tuple(jnp.zeros((16,), jnp.float32) for _ in range(NV)))
        carry = lax.fori_loop(n4, hi, row_body, carry)
        for t in range(NV):
          ostage[s, pl.ds(16 * t, 16)] += carry[t]

  def pair_body(j, _):
    g0 = 2 * j
    chunk_start(g0 + 1, 1)
    chunk_wait(g0, 0)
    chunk_rows(g0, 0)
    chunk_start(g0 + 2, 0)
    chunk_wait(g0 + 1, 1)
    chunk_rows(g0 + 1, 1)
    return 0

  chunk_start(0, 0)
  lax.fori_loop(0, lax.div(nch + 1, 2), pair_body, 0)

  pltpu.sync_copy(ostage, fsum_hbm.at[wid])


BN2 = 4000
NB2 = N // BN2


def _tc_aux_body(ids_ref, c3_ref, aux_ref):
  i = pl.program_id(0)

  @pl.when(i == 0)
  def _():
    aux_ref[...] = jnp.zeros_like(aux_ref)

  ids = ids_ref[0, 0]
  oh = (lax.broadcasted_iota(jnp.int32, (K, BN2), 0)
        == ids[None, :]).astype(jnp.float32)
  csum = lax.dot_general(oh, c3_ref[...], (((1,), (0,)), ((), ())),
                         preferred_element_type=jnp.float32)
  cnt = jnp.sum(oh, axis=1, keepdims=True)
  aux_ref[...] += jnp.concatenate([csum, cnt], axis=1)


def _tc_head_body(aux_ref, fs_ref, w1_ref, w2_ref, w3_ref, b3_ref,
                  emb_ref, cent_ref, out_ref):
  aux4 = aux_ref[...]
  inv = 1.0 / jnp.maximum(aux4[:, 3:4], 1.0)
  emb = fs_ref[...] * inv
  emb_ref[...] = emb
  cent_ref[...] = aux4[:, 0:3] * inv
  h = jax.nn.relu(jnp.dot(emb, w1_ref[...],
                          preferred_element_type=jnp.float32))
  h = jax.nn.relu(jnp.dot(h, w2_ref[...],
                          preferred_element_type=jnp.float32))
  out_ref[...] = (jnp.dot(h, w3_ref[...],
                          preferred_element_type=jnp.float32) + b3_ref[...])


def kernel(features, coords, instance_ids, W1, W2, W3, b3):
  ids = instance_ids.astype(jnp.int32)

  fsum = _sc_segment_sum(features, ids, ids[::DEC])

  aux4 = pl.pallas_call(
      _tc_aux_body,
      grid=(NB2,),
      in_specs=[
          pl.BlockSpec((1, 1, BN2), lambda i: (i, 0, 0)),
          pl.BlockSpec((BN2, 3), lambda i: (i, 0)),
      ],
      out_specs=pl.BlockSpec((K, 4), lambda i: (0, 0)),
      out_shape=jax.ShapeDtypeStruct((K, 4), jnp.float32),
  )(ids.reshape(NB2, 1, BN2), coords.astype(jnp.float32))

  emb, cent, out = pl.pallas_call(
      _tc_head_body,
      out_shape=[jax.ShapeDtypeStruct((K, D), jnp.float32),
                 jax.ShapeDtypeStruct((K, 3), jnp.float32),
                 jax.ShapeDtypeStruct((K, OUT), jnp.float32)],
  )(aux4, fsum.reshape(K, D), W1, W2, W3, b3)
  return emb, cent, out
